# baseline (speedup 1.0000x reference)
import functools

import jax
import jax.numpy as jnp
from jax import lax
from jax.experimental import pallas as pl
from jax.experimental.pallas import tpu as pltpu
from jax.experimental.pallas import tpu_sc as plsc

_VOCAB1 = 100001
_D = 128
_B = 4096
_L = 200
_LC = 100
_NC = 2
_NS = 16
_NW = _NC * _NS
_BPW = _B // _NW
_NG = _BPW // 16
_NV = _D // 16


def _sc_pooled_lookup(kw2, lens, table):
  mesh = plsc.VectorSubcoreMesh(
      core_axis_name="c", subcore_axis_name="s", num_cores=_NC,
      num_subcores=_NS)

  @functools.partial(
      pl.kernel,
      out_type=jax.ShapeDtypeStruct((_B, _D), jnp.float32),
      mesh=mesh,
      scratch_types=[
          pltpu.VMEM((2 * _BPW, _LC), jnp.int32),
          pltpu.VMEM((2, _L, _D), jnp.float32),
          pltpu.VMEM((_BPW, _D), jnp.float32),
          pltpu.VMEM((_BPW,), jnp.int32),
          pltpu.VMEM((_BPW,), jnp.float32),
          pltpu.SemaphoreType.DMA,
          pltpu.SemaphoreType.DMA,
      ],
  )
  def kernel_body(kw_hbm, len_hbm, table_hbm, out_hbm,
                  idx_v, rows_v, out_v, len_v, recip_v, sem0, sem1):
    sems = (sem0, sem1)
    wid = lax.axis_index("s") * _NC + lax.axis_index("c")
    base = wid * _BPW

    pltpu.sync_copy(kw_hbm.at[pl.ds(base * 2, 2 * _BPW)], idx_v)
    pltpu.sync_copy(len_hbm.at[pl.ds(base, _BPW)], len_v)

    for j in range(_NG):
      lv = len_v[pl.ds(j * 16, 16)]
      lf = jnp.maximum(lv.astype(jnp.float32), 1.0)
      recip_v[pl.ds(j * 16, 16)] = 1.0 / lf

    def issue_gather(b, buf):
      pltpu.async_copy(table_hbm.at[idx_v.at[2 * b]],
                       rows_v.at[buf, pl.ds(0, _LC)], sems[buf])
      pltpu.async_copy(table_hbm.at[idx_v.at[2 * b + 1]],
                       rows_v.at[buf, pl.ds(_LC, _LC)], sems[buf])

    def wait_gather(buf):
      pltpu.make_async_copy(table_hbm.at[pl.ds(0, _L)], rows_v.at[buf],
                            sems[buf]).wait()

    issue_gather(jnp.int32(0), 0)

    def group_body(j, carry):
      rchunk = recip_v[pl.ds(j * 16, 16)]
      for k in range(16):
        b = j * 16 + k
        buf = k % 2

        nb = b + 1

        @pl.when(nb < _BPW)
        def _():
          issue_gather(nb, (k + 1) % 2)

        wait_gather(buf)

        def red_body(r, acc):
          r0 = r * 4
          for u in range(4):
            acc = tuple(acc[d] + rows_v[buf, r0 + u, pl.ds(d * 16, 16)]
                        for d in range(_NV))
          return acc

        acc = lax.fori_loop(
            0, _L // 4, red_body,
            tuple(jnp.zeros((16,), jnp.float32) for _ in range(_NV)))

        rk = jnp.broadcast_to(lax.slice(rchunk, (k,), (k + 1,)), (16,))
        for d in range(_NV):
          out_v[b, pl.ds(d * 16, 16)] = acc[d] * rk
      return carry

    lax.fori_loop(0, _NG, group_body, 0)

    pltpu.sync_copy(out_v, out_hbm.at[pl.ds(base, _BPW)])

  return kernel_body(kw2, lens, table)


@jax.jit
def kernel(keyword_lists, keyword_lengths, embedding_weight):
  kw2 = keyword_lists.reshape(_B * 2, _LC)
  lens = keyword_lengths.reshape(_B)
  return _sc_pooled_lookup(kw2, lens, embedding_weight)


# speedup vs baseline: 1.2300x; 1.2300x over previous
import functools

import jax
import jax.numpy as jnp
from jax import lax
from jax.experimental import pallas as pl
from jax.experimental.pallas import tpu as pltpu
from jax.experimental.pallas import tpu_sc as plsc

_VOCAB1 = 100001
_D = 128
_B = 4096
_L = 200
_LC = 100
_NC = 2
_NS = 16
_NW = _NC * _NS
_BPW = _B // _NW
_NU = 2 * _BPW
_NG = _BPW // 16
_NV = _D // 16
_NSLOT = 4


def _sc_pooled_lookup(kw2, lens, table):
  mesh = plsc.VectorSubcoreMesh(
      core_axis_name="c", subcore_axis_name="s", num_cores=_NC,
      num_subcores=_NS)

  @functools.partial(
      pl.kernel,
      out_type=jax.ShapeDtypeStruct((_B * _D,), jnp.float32),
      mesh=mesh,
      scratch_types=[
          pltpu.VMEM((_NU, _LC), jnp.int32),
          pltpu.VMEM((_NSLOT, _LC, _D), jnp.float32),
          pltpu.VMEM((_BPW * _D,), jnp.float32),
          pltpu.VMEM((_BPW,), jnp.int32),
          pltpu.VMEM((_BPW,), jnp.float32),
          [pltpu.SemaphoreType.DMA] * _NSLOT,
      ],
  )
  def kernel_body(kw_hbm, len_hbm, table_hbm, out_hbm,
                  idx_v, rows_v, out_v, len_v, recip_v, sems):
    wid = lax.axis_index("s") * _NC + lax.axis_index("c")
    base = wid * _BPW

    pltpu.sync_copy(kw_hbm.at[pl.ds(base * 2, _NU)], idx_v)
    pltpu.sync_copy(len_hbm.at[pl.ds(base, _BPW)], len_v)

    for j in range(_NG):
      lv = len_v[pl.ds(j * 16, 16)]
      lf = jnp.maximum(lv.astype(jnp.float32), 1.0)
      recip_v[pl.ds(j * 16, 16)] = 1.0 / lf

    def issue_unit(u, slot):
      pltpu.async_copy(table_hbm.at[idx_v.at[u]], rows_v.at[slot],
                       sems[slot])

    def wait_unit(u, slot):
      pltpu.make_async_copy(table_hbm.at[idx_v.at[u]], rows_v.at[slot],
                            sems[slot]).wait()

    for s in range(_NSLOT - 1):
      issue_unit(jnp.int32(s), s)

    def group_body(j, carry):
      rchunk = recip_v[pl.ds(j * 16, 16)]
      acc = None
      for k in range(32):
        u = j * 32 + k
        slot = k % _NSLOT

        nu = u + (_NSLOT - 1)

        @pl.when(nu < _NU)
        def _():
          issue_unit(nu, (k + _NSLOT - 1) % _NSLOT)

        wait_unit(u, slot)

        if k % 2 == 0:
          acc = tuple(jnp.zeros((16,), jnp.float32) for _ in range(_NV))

        def red_body(r, a):
          return tuple(a[d] + rows_v[slot, r, pl.ds(d * 16, 16)]
                       for d in range(_NV))

        acc = lax.fori_loop(0, _LC, red_body, acc)

        if k % 2 == 1:
          kb = k // 2
          b = j * 16 + kb
          rk = jnp.broadcast_to(lax.slice(rchunk, (kb,), (kb + 1,)), (16,))
          rowbase = b * _D
          for d in range(_NV):
            out_v[pl.ds(rowbase + d * 16, 16)] = acc[d] * rk
      return carry

    lax.fori_loop(0, _NG, group_body, 0)

    pltpu.sync_copy(out_v, out_hbm.at[pl.ds(base * _D, _BPW * _D)])

  return kernel_body(kw2, lens, table)


@jax.jit
def kernel(keyword_lists, keyword_lengths, embedding_weight):
  kw2 = keyword_lists.reshape(_B * 2, _LC)
  lens = keyword_lengths.reshape(_B)
  return _sc_pooled_lookup(kw2, lens, embedding_weight).reshape(_B, _D)
